# Initial kernel scaffold; baseline (speedup 1.0000x reference)
import jax
import jax.numpy as jnp
from jax.experimental import pallas as pl


def kernel(feature_values, feature_idx, feature_embeddings):
    raise NotImplementedError("write your pallas kernel here")


# speedup vs baseline: 1.9681x; 1.9681x over previous
import functools

import jax
import jax.numpy as jnp
from jax import lax
from jax.experimental import pallas as pl
from jax.experimental.pallas import tpu as pltpu
from jax.experimental.pallas import tpu_sc as plsc

B = 16384
F = 26
K = 32
NC = 2
NS = 16
NW = NC * NS
BPW = B // NW
C = 64
NCHUNK = BPW // C
L = 16

_mesh = plsc.VectorSubcoreMesh(core_axis_name="c", subcore_axis_name="s")


@functools.partial(
    pl.kernel,
    out_type=jax.ShapeDtypeStruct((B, K), jnp.float32),
    mesh=_mesh,
    compiler_params=pltpu.CompilerParams(use_tc_tiling_on_sc=False),
    scratch_types=[
        pltpu.VMEM((C * F,), jnp.int32),
        pltpu.VMEM((C, K), jnp.float32),
        pltpu.VMEM((C * F, K), jnp.float32),
        pltpu.VMEM((C, K), jnp.float32),
        pltpu.SemaphoreType.DMA,
    ],
)
def _fm_second_order(vals_hbm, idx_hbm, table_hbm, out_hbm,
                     idx_v, vals_v, rows_v, out_v, sem):
    wid = lax.axis_index("s") * NC + lax.axis_index("c")
    base = wid * BPW

    def chunk_body(g, carry):
        row0 = base + g * C
        flat0 = row0 * F
        pltpu.sync_copy(idx_hbm.at[pl.ds(flat0, C * F)], idx_v)
        pltpu.sync_copy(vals_hbm.at[pl.ds(row0, C)], vals_v)
        pltpu.async_copy(table_hbm.at[idx_v], rows_v, sem).wait()

        def b_body(b, carry2):
            a1_lo = jnp.zeros((L,), jnp.float32)
            a1_hi = jnp.zeros((L,), jnp.float32)
            a2_lo = jnp.zeros((L,), jnp.float32)
            a2_hi = jnp.zeros((L,), jnp.float32)
            j0 = b * F
            v_lo = vals_v[b, pl.ds(0, L)]
            v_hi = vals_v[b, pl.ds(L, L)]
            for f in range(F):
                w = v_lo[f] if f < L else v_hi[f - L]
                e_lo = rows_v[j0 + f, pl.ds(0, L)]
                e_hi = rows_v[j0 + f, pl.ds(L, L)]
                we_lo = w * e_lo
                we_hi = w * e_hi
                a1_lo = a1_lo + we_lo
                a1_hi = a1_hi + we_hi
                a2_lo = a2_lo + we_lo * we_lo
                a2_hi = a2_hi + we_hi * we_hi
            out_v[b, pl.ds(0, L)] = a1_lo * a1_lo - a2_lo
            out_v[b, pl.ds(L, L)] = a1_hi * a1_hi - a2_hi
            return carry2

        lax.fori_loop(0, C, b_body, 0)
        pltpu.sync_copy(out_v, out_hbm.at[pl.ds(row0, C)])
        return carry

    lax.fori_loop(0, NCHUNK, chunk_body, 0)


def kernel(feature_values, feature_idx, feature_embeddings):
    vals_padded = jnp.pad(feature_values, ((0, 0), (0, K - F)))
    return _fm_second_order(
        vals_padded,
        feature_idx.reshape(-1),
        feature_embeddings,
    )
